# baseline (speedup 1.0000x reference)
import functools

import jax
import jax.numpy as jnp
from jax import lax
from jax.experimental import pallas as pl
from jax.experimental.pallas import tpu as pltpu
from jax.experimental.pallas import tpu_sc as plsc

_BATCH = 4
_ROWS = 8192
_DIM = 1024
_NC = 2
_NS = 16
_NW = _NC * _NS
_ROWS_PER_W = _ROWS // _NW
_CHUNK = 64
_STEPS = _ROWS_PER_W // _CHUNK

_mesh = plsc.VectorSubcoreMesh(core_axis_name="c", subcore_axis_name="s")


@functools.partial(
    pl.kernel,
    mesh=_mesh,
    out_type=jax.ShapeDtypeStruct((_BATCH * _ROWS, _DIM), jnp.float32),
    scratch_types=[pltpu.VMEM((_CHUNK, _DIM), jnp.float32)],
)
def _broadcast_table(table_hbm, out_hbm, buf):
    wid = lax.axis_index("s") * _NC + lax.axis_index("c")
    base = wid * _ROWS_PER_W
    for b in range(_BATCH):
        pltpu.sync_copy(
            table_hbm.at[pl.ds(base, _ROWS_PER_W)],
            out_hbm.at[pl.ds(b * _ROWS + base, _ROWS_PER_W)],
        )


def kernel(x, table):
    del x
    flat = _broadcast_table(table)
    return flat.reshape(_BATCH, _ROWS, _DIM)


# speedup vs baseline: 54.5789x; 54.5789x over previous
import functools

import jax
import jax.numpy as jnp
from jax import lax
from jax.experimental import pallas as pl
from jax.experimental.pallas import tpu as pltpu
from jax.experimental.pallas import tpu_sc as plsc

_BATCH = 4
_ROWS = 8192
_DIM = 1024
_NC = 2
_NS = 16
_NW = _NC * _NS
_ROWS_PER_W = _ROWS // _NW
_CHUNK = 32
_STEPS = _ROWS_PER_W // _CHUNK
_NBUF = 3

_mesh = plsc.VectorSubcoreMesh(core_axis_name="c", subcore_axis_name="s")


@functools.partial(
    pl.kernel,
    mesh=_mesh,
    out_type=jax.ShapeDtypeStruct((_BATCH * _ROWS, _DIM), jnp.float32),
    scratch_types=(
        [pltpu.VMEM((_CHUNK, _DIM), jnp.float32)] * _NBUF
        + [pltpu.SemaphoreType.DMA] * (2 * _NBUF)
    ),
)
def _broadcast_table(table_hbm, out_hbm, *scratch):
    bufs = scratch[:_NBUF]
    rsems = scratch[_NBUF:2 * _NBUF]
    wsems = scratch[2 * _NBUF:]
    wid = lax.axis_index("s") * _NC + lax.axis_index("c")
    base = wid * _ROWS_PER_W

    def start_read(s):
        k = s % _NBUF
        return pltpu.async_copy(
            table_hbm.at[pl.ds(base + s * _CHUNK, _CHUNK)], bufs[k], rsems[k])

    reads = {s: start_read(s) for s in range(min(_NBUF, _STEPS))}
    for s in range(_STEPS):
        k = s % _NBUF
        reads[s].wait()
        writes = [
            pltpu.async_copy(
                bufs[k], out_hbm.at[pl.ds(b * _ROWS + base + s * _CHUNK, _CHUNK)],
                wsems[k])
            for b in range(_BATCH)
        ]
        for w in writes:
            w.wait()
        if s + _NBUF < _STEPS:
            reads[s + _NBUF] = start_read(s + _NBUF)


def kernel(x, table):
    del x
    flat = _broadcast_table(table)
    return flat.reshape(_BATCH, _ROWS, _DIM)


# speedup vs baseline: 55.4904x; 1.0167x over previous
import functools

import jax
import jax.numpy as jnp
from jax import lax
from jax.experimental import pallas as pl
from jax.experimental.pallas import tpu as pltpu
from jax.experimental.pallas import tpu_sc as plsc

_BATCH = 4
_ROWS = 8192
_DIM = 1024
_NC = 2
_NS = 16
_NW = _NC * _NS
_ROWS_PER_W = _ROWS // _NW
_CHUNK = 64
_STEPS = _ROWS_PER_W // _CHUNK

_mesh = plsc.VectorSubcoreMesh(core_axis_name="c", subcore_axis_name="s")


@functools.partial(
    pl.kernel,
    mesh=_mesh,
    out_type=jax.ShapeDtypeStruct((_BATCH * _ROWS, _DIM), jnp.float32),
    scratch_types=[pltpu.VMEM((_CHUNK, _DIM), jnp.float32)],
)
def _broadcast_table(table_hbm, out_hbm, buf):
    wid = lax.axis_index("s") * _NC + lax.axis_index("c")
    base = wid * _ROWS_PER_W
    for s in range(_STEPS):
        r = base + s * _CHUNK
        pltpu.sync_copy(table_hbm.at[pl.ds(r, _CHUNK)], buf)
        for b in range(_BATCH):
            pltpu.sync_copy(buf, out_hbm.at[pl.ds(b * _ROWS + r, _CHUNK)])


def kernel(x, table):
    del x
    flat = _broadcast_table(table)
    return flat.reshape(_BATCH, _ROWS, _DIM)
